# Initial kernel scaffold; baseline (speedup 1.0000x reference)
import jax
import jax.numpy as jnp
from jax.experimental import pallas as pl


def kernel(x, edges, edge_weights, edge_attrs, bbox_idx, head_Wm, head_Wa, head_Wn, head_Wo, head_bo, head_Ws, head_bs, blk_Wm, blk_Wa, blk_Wn, blk_Wo, blk_bo, blk_Ws, blk_bs, fus_W, fus_b, fuss_W, fuss_b):
    raise NotImplementedError("write your pallas kernel here")


# speedup vs baseline: 2.3131x; 2.3131x over previous
import functools

import jax
import jax.numpy as jnp
from jax import lax
from jax.experimental import pallas as pl
from jax.experimental.pallas import tpu as pltpu
from jax.experimental.pallas import tpu_sc as plsc

_N = 10000
_NP = 10112
_E = 320000
_C = 128
_NSUPER = 1024

_NC = 2
_NS = 16
_NW = _NC * _NS
_K = 128
_SUP = 8


@functools.partial(jax.jit, static_argnames=("n_out", "d", "scale",
                                             "with_ea"))
def _sc_segsum(table, idx3, dst3, ew3=None, ea4=None, *, n_out, d, scale,
               with_ea=False):
  n_chunks = idx3.shape[1]
  sup = _SUP if n_chunks % _SUP == 0 else n_chunks
  n_super = n_chunks // sup
  spr = n_out // _NS
  zr = 8
  assert spr % zr == 0 and d % _C == 0

  mesh = plsc.VectorSubcoreMesh(
      core_axis_name="c", subcore_axis_name="s", num_cores=_NC,
      num_subcores=_NS)

  scratch = [
      pltpu.VMEM((sup, _K), jnp.int32),
      pltpu.VMEM((sup, _K), jnp.int32),
      pltpu.VMEM((sup, _K), jnp.float32) if scale else None,
      pltpu.VMEM((_K, 16), jnp.float32) if with_ea else None,
      pltpu.VMEM((_K, d), jnp.float32),
      pltpu.VMEM((zr, d), jnp.float32),
      pltpu.VMEM_SHARED((n_out, d), jnp.float32),
      pltpu.SemaphoreType.DMA,
      pltpu.SemaphoreType.DMA,
  ]
  scratch = [sc for sc in scratch if sc is not None]

  @functools.partial(
      pl.kernel, mesh=mesh,
      out_type=jax.ShapeDtypeStruct((_NC, n_out, d), jnp.float32),
      scratch_types=scratch)
  def k(*refs):
    refs = list(refs)
    table_h = refs.pop(0)
    idx_h = refs.pop(0)
    dst_h = refs.pop(0)
    ew_h = refs.pop(0) if scale else None
    ea_h = refs.pop(0) if with_ea else None
    out_h = refs.pop(0)
    idxs_v = refs.pop(0)
    dsts_v = refs.pop(0)
    ews_v = refs.pop(0) if scale else None
    eab_v = refs.pop(0) if with_ea else None
    rows_v, zbuf_v, acc_sh, sem, zsem = refs

    c = lax.axis_index("c")
    s = lax.axis_index("s")
    w = c * _NS + s

    z16 = jnp.zeros((16,), jnp.float32)
    for zi in range(zr):
      for j in range(d // 16):
        zbuf_v[zi, pl.ds(j * 16, 16)] = z16
    handles = [
        pltpu.async_copy(zbuf_v, acc_sh.at[pl.ds(s * spr + t * zr, zr)], zsem)
        for t in range(spr // zr)
    ]
    for h in handles:
      h.wait()
    plsc.subcore_barrier()

    def superchunk(sc_i, carry):
      pltpu.sync_copy(idx_h.at[w, pl.ds(sc_i * sup, sup)], idxs_v)
      pltpu.sync_copy(dst_h.at[w, pl.ds(sc_i * sup, sup)], dsts_v)
      if scale:
        pltpu.sync_copy(ew_h.at[w, pl.ds(sc_i * sup, sup)], ews_v)
      for j in range(sup):
        pltpu.async_copy(table_h.at[idxs_v.at[j]], rows_v, sem).wait()
        if with_ea:
          pltpu.sync_copy(ea_h.at[w, sc_i * sup + j], eab_v)
        if scale:
          def srow16(rb, cc):
            ew16 = ews_v[j, pl.ds(rb * 16, 16)]
            for ln in range(16):
              w16 = ew16.at[jnp.full((16,), ln, jnp.int32)].get(
                  mode="promise_in_bounds")
              r = rb * 16 + ln
              if with_ea:
                rows_v[r, pl.ds(16, 16)] = eab_v[r, pl.ds(0, 16)]
              for jj in range(d // 16):
                sl = pl.ds(jj * 16, 16)
                rows_v[r, sl] = rows_v[r, sl] * w16
            return cc
          lax.fori_loop(0, _K // 16, srow16, 0)
        pltpu.sync_copy(rows_v, acc_sh.at[dsts_v.at[j]], add=True)
      return carry

    lax.fori_loop(0, n_super, superchunk, 0)
    plsc.subcore_barrier()
    pltpu.sync_copy(acc_sh.at[pl.ds(s * spr, spr)],
                    out_h.at[c, pl.ds(s * spr, spr)])

  args = [table, idx3, dst3]
  if scale:
    args.append(ew3)
  if with_ea:
    args.append(ea4)
  return k(*args)


def _tc_conv_block(p, a, hn, fp, fsp, Wm, Wa, Wn, Wo, bo, Ws, bs, *, head):
  n = p.shape[1]
  din = p.shape[2]
  bn = n // 8
  grid = (n // bn,)

  row = lambda shp: pl.BlockSpec(shp, lambda i: (i, 0))
  full = lambda shp: pl.BlockSpec(shp, lambda i: (0, 0))
  in_specs = [
      pl.BlockSpec((1, bn, din), lambda i: (0, i, 0)),
      pl.BlockSpec((1, bn, din), lambda i: (1, i, 0)),
      row((bn, hn.shape[1])),
  ]
  args = [p, p, hn]
  if not head:
    in_specs += [
        pl.BlockSpec((1, bn, _C), lambda i: (0, i, 0)),
        pl.BlockSpec((1, bn, _C), lambda i: (1, i, 0)),
        row((bn, _C)), row((bn, _C)),
    ]
    args += [a, a, fp, fsp]
  if not head:
    in_specs += [full(Wm.shape), full(Wa.shape)]
    args += [Wm, Wa]
  else:
    in_specs += [full(Wm.shape)]
    args += [Wm]
  in_specs += [full(Wn.shape), full(Wo.shape), full((1, _C)),
               full(Ws.shape), full((1, _C))]
  args += [Wn, Wo, bo.reshape(1, _C), Ws, bs.reshape(1, _C)]

  def body(*refs):
    if head:
      (p0, p1, hn_r, W0_r, Wn_r, Wo_r, bo_r, Ws_r, bs_r, f_o, fs_o) = refs
      g = p0[0] + p1[0]
      agg = jnp.dot(g, W0_r[...], preferred_element_type=jnp.float32)
    else:
      (p0, p1, hn_r, a0, a1, fp_r, fsp_r, Wm_r, Wa_r, Wn_r, Wo_r, bo_r,
       Ws_r, bs_r, f_o, fs_o) = refs
      g = p0[0] + p1[0]
      av = a0[0] + a1[0]
      agg = (jnp.dot(g, Wm_r[...], preferred_element_type=jnp.float32)
             + jnp.dot(av, Wa_r[...], preferred_element_type=jnp.float32))
    node = jnp.dot(hn_r[...], Wn_r[...], preferred_element_type=jnp.float32)
    f = jnp.maximum(
        jnp.dot(agg, Wo_r[...], preferred_element_type=jnp.float32)
        + node + bo_r[...], 0.0)
    fs = jnp.maximum(
        jnp.dot(agg + node, Ws_r[...], preferred_element_type=jnp.float32)
        + bs_r[...], 0.0)
    if not head:
      f = f + fp_r[...]
      fs = fs + fsp_r[...]
    f_o[...] = f
    fs_o[...] = fs

  return pl.pallas_call(
      body,
      grid=grid,
      in_specs=in_specs,
      out_specs=[row((bn, _C)), row((bn, _C))],
      out_shape=[jax.ShapeDtypeStruct((n, _C), jnp.float32)] * 2,
  )(*args)


def _tc_fusion(f5, f6, W, b):
  n = f5.shape[0]
  bn = n // 8
  row = lambda shp: pl.BlockSpec(shp, lambda i: (i, 0))
  full = lambda shp: pl.BlockSpec(shp, lambda i: (0, 0))

  def body(f5_r, f6_r, W_r, b_r, o_r):
    fcat = jnp.concatenate([f5_r[...], f6_r[...]], axis=1)
    o_r[...] = jnp.maximum(
        jnp.dot(fcat, W_r[...], preferred_element_type=jnp.float32)
        + b_r[...], 0.0)

  return pl.pallas_call(
      body,
      grid=(n // bn,),
      in_specs=[row((bn, _C)), row((bn, _C)), full(W.shape), full((1, 1024))],
      out_specs=row((bn, 1024)),
      out_shape=jax.ShapeDtypeStruct((n, 1024), jnp.float32),
  )(f5, f6, W, b.reshape(1, 1024))


def _tc_super(p5, p6, pc, W, b):
  def body(p50, p51, p60, p61, pc0, pc1, W_r, b_r, mean_o, fus_o):
    cnt = jnp.maximum(pc0[0][:, 0:1] + pc1[0][:, 0:1], 1.0)
    inv = 1.0 / cnt
    m5 = (p50[0] + p51[0]) * inv
    m6 = (p60[0] + p61[0]) * inv
    mean = jnp.concatenate([m5, m6], axis=1)
    mean_o[...] = mean
    fus_o[...] = jnp.maximum(
        jnp.dot(mean, W_r[...], preferred_element_type=jnp.float32)
        + b_r[...], 0.0)

  part = lambda cc: pl.BlockSpec((1, _NSUPER, _C), lambda i, cc=cc: (cc, 0, 0))
  full = lambda shp: pl.BlockSpec(shp, lambda i: (0,) * len(shp))
  return pl.pallas_call(
      body,
      grid=(1,),
      in_specs=[part(0), part(1), part(0), part(1), part(0), part(1),
                full(W.shape), full((1, 1024))],
      out_specs=[
          full((_NSUPER, 256)),
          full((_NSUPER, 1024)),
      ],
      out_shape=[
          jax.ShapeDtypeStruct((_NSUPER, 256), jnp.float32),
          jax.ShapeDtypeStruct((_NSUPER, 1024), jnp.float32),
      ],
  )(p5, p5, p6, p6, pc, pc, W, b.reshape(1, 1024))


def _pad_rows(v, r):
  return jnp.pad(v, ((0, r - v.shape[0]),) + ((0, 0),) * (v.ndim - 1))


def kernel(x, edges, edge_weights, edge_attrs, bbox_idx, head_Wm, head_Wa,
           head_Wn, head_Wo, head_bo, head_Ws, head_bs, blk_Wm, blk_Wa,
           blk_Wn, blk_Wo, blk_bo, blk_Ws, blk_bs, fus_W, fus_b, fuss_W,
           fuss_b):
  src = edges[0, 0]
  dst = edges[0, 1]
  ew = edge_weights[0]
  ea = edge_attrs[0]

  gran = _NW * _K * _SUP
  epad = ((_E + gran - 1) // gran) * gran
  src3 = _pad_rows(src, epad).reshape(_NW, -1, _K)
  dst3 = _pad_rows(dst, epad).reshape(_NW, -1, _K)
  ew3 = _pad_rows(ew, epad).reshape(_NW, -1, _K)

  x_pad128 = _pad_rows(jnp.pad(x, ((0, 0), (0, _C - x.shape[1]))), _NP)
  x_pad16 = x_pad128[:, :16]
  ea4 = _pad_rows(jnp.pad(ea, ((0, 0), (0, 16 - ea.shape[1]))),
                  epad).reshape(_NW, -1, _K, 16)

  ga_p = _sc_segsum(x_pad128, src3, dst3, ew3, ea4, n_out=_NP, d=_C,
                    scale=True, with_ea=True)

  w0 = (jnp.pad(head_Wm, ((0, _C - head_Wm.shape[0]), (0, 0)))
        + jnp.pad(head_Wa, ((16, _C - 16 - head_Wa.shape[0]), (0, 0))))
  wn0 = jnp.pad(head_Wn, ((0, 16 - head_Wn.shape[0]), (0, 0)))
  f, fs = _tc_conv_block(ga_p, None, x_pad16, None, None, w0, None, wn0,
                         head_Wo, head_bo, head_Ws, head_bs, head=True)

  wa_e = jnp.pad(blk_Wa, ((0, 0), (16, _C - 16 - blk_Wa.shape[1]), (0, 0)))
  feats = [f]
  feats_super = [fs]
  for i in range(6):
    g_p = _sc_segsum(feats[-1], src3, dst3, ew3, n_out=_NP, d=_C, scale=True)
    nf, nfs = _tc_conv_block(
        g_p, ga_p, feats_super[-1], feats[-1], feats_super[-1],
        blk_Wm[i], wa_e[i], blk_Wn[i], blk_Wo[i], blk_bo[i], blk_Ws[i],
        blk_bs[i], head=False)
    feats.append(nf)
    feats_super.append(nfs)

  f5, f6 = feats[5], feats[6]
  fs5, fs6 = feats_super[5], feats_super[6]

  fusion = _tc_fusion(f5, f6, fus_W, fus_b)
  out_feat = jnp.concatenate([fusion[:_N], f5[:_N], f6[:_N]], axis=1)

  npad = ((_N + _NW * _K - 1) // (_NW * _K)) * (_NW * _K)
  nsp = 1152
  bidx3 = jnp.pad(bbox_idx.astype(jnp.int32), (0, npad - _N),
                  constant_values=_NSUPER).reshape(_NW, -1, _K)
  niota3 = jnp.minimum(jnp.arange(npad, dtype=jnp.int32),
                       _N - 1).reshape(_NW, -1, _K)
  ones_tab = jnp.ones((_N, _C), jnp.float32)
  p5 = _sc_segsum(fs5, niota3, bidx3, n_out=nsp, d=_C, scale=False)
  p6 = _sc_segsum(fs6, niota3, bidx3, n_out=nsp, d=_C, scale=False)
  pc = _sc_segsum(ones_tab, niota3, bidx3, n_out=nsp, d=_C, scale=False)

  mean, fusion_s = _tc_super(p5, p6, pc, fuss_W, fuss_b)
  out_feat_super = jnp.concatenate([fusion_s, mean], axis=1)
  return (out_feat, out_feat_super)


# speedup vs baseline: 2.6198x; 1.1326x over previous
import functools

import jax
import jax.numpy as jnp
from jax import lax
from jax.experimental import pallas as pl
from jax.experimental.pallas import tpu as pltpu
from jax.experimental.pallas import tpu_sc as plsc

_N = 10000
_NP = 10112
_E = 320000
_C = 128
_NSUPER = 1024

_NC = 2
_NS = 16
_NW = _NC * _NS
_K = 128
_SUP = 8


@functools.partial(jax.jit, static_argnames=("n_out", "d", "scale",
                                             "with_ea"))
def _sc_segsum(table, idx3, dst3, ew3=None, ea4=None, *, n_out, d, scale,
               with_ea=False):
  n_chunks = idx3.shape[1]
  sup = _SUP if n_chunks % _SUP == 0 else n_chunks
  n_super = n_chunks // sup
  spr = n_out // _NS
  zr = 8
  assert spr % zr == 0 and d % _C == 0

  mesh = plsc.VectorSubcoreMesh(
      core_axis_name="c", subcore_axis_name="s", num_cores=_NC,
      num_subcores=_NS)

  scratch = [
      pltpu.VMEM((sup, _K), jnp.int32),
      pltpu.VMEM((sup, _K), jnp.int32),
      pltpu.VMEM((sup, _K), jnp.float32) if scale else None,
      pltpu.VMEM((16, _C), jnp.float32) if with_ea else None,
      pltpu.VMEM((_K, d), jnp.float32),
      pltpu.VMEM((_K, d), jnp.float32),
      pltpu.VMEM((zr, d), jnp.float32),
      pltpu.VMEM_SHARED((n_out, d), jnp.float32),
      pltpu.SemaphoreType.DMA,
      pltpu.SemaphoreType.DMA,
      pltpu.SemaphoreType.DMA,
  ]
  scratch = [sc for sc in scratch if sc is not None]

  @functools.partial(
      pl.kernel, mesh=mesh,
      out_type=jax.ShapeDtypeStruct((_NC, n_out, d), jnp.float32),
      scratch_types=scratch)
  def k(*refs):
    refs = list(refs)
    table_h = refs.pop(0)
    idx_h = refs.pop(0)
    dst_h = refs.pop(0)
    ew_h = refs.pop(0) if scale else None
    ea_h = refs.pop(0) if with_ea else None
    out_h = refs.pop(0)
    idxs_v = refs.pop(0)
    dsts_v = refs.pop(0)
    ews_v = refs.pop(0) if scale else None
    eab_v = refs.pop(0) if with_ea else None
    rows_v, rows2_v, zbuf_v, acc_sh, sem, ssem, zsem = refs
    bufs = (rows_v, rows2_v)

    c = lax.axis_index("c")
    s = lax.axis_index("s")
    w = c * _NS + s

    z16 = jnp.zeros((16,), jnp.float32)
    for zi in range(zr):
      for j in range(d // 16):
        zbuf_v[zi, pl.ds(j * 16, 16)] = z16
    handles = [
        pltpu.async_copy(zbuf_v, acc_sh.at[pl.ds(s * spr + t * zr, zr)], zsem)
        for t in range(spr // zr)
    ]
    for h in handles:
      h.wait()
    plsc.subcore_barrier()

    def superchunk(sc_i, carry):
      pltpu.sync_copy(idx_h.at[w, pl.ds(sc_i * sup, sup)], idxs_v)
      pltpu.sync_copy(dst_h.at[w, pl.ds(sc_i * sup, sup)], dsts_v)
      if scale:
        pltpu.sync_copy(ew_h.at[w, pl.ds(sc_i * sup, sup)], ews_v)
      gat = [None] * sup
      sca = [None] * sup
      gat[0] = pltpu.async_copy(table_h.at[idxs_v.at[0]], bufs[0], sem)
      for j in range(sup):
        buf = bufs[j % 2]
        if j < sup - 1:
          if j >= 1:
            sca[j - 1].wait()
          gat[j + 1] = pltpu.async_copy(
              table_h.at[idxs_v.at[j + 1]], bufs[(j + 1) % 2], sem)
        gat[j].wait()
        if with_ea:
          pltpu.sync_copy(ea_h.at[w, sc_i * sup + j], eab_v)
        if scale:
          def srow16(rb, cc, buf=buf, j=j):
            ew16 = ews_v[j, pl.ds(rb * 16, 16)]
            for ln in range(16):
              w16 = ew16.at[jnp.full((16,), ln, jnp.int32)].get(
                  mode="promise_in_bounds")
              r = rb * 16 + ln
              if with_ea:
                buf[r, pl.ds(16, 16)] = eab_v[rb * 2 + ln // 8,
                                              pl.ds((ln % 8) * 16, 16)]
              for jj in range(d // 16):
                sl = pl.ds(jj * 16, 16)
                buf[r, sl] = buf[r, sl] * w16
            return cc
          lax.fori_loop(0, _K // 16, srow16, 0)
        sca[j] = pltpu.async_copy(buf, acc_sh.at[dsts_v.at[j]], ssem,
                                  add=True)
      if sup >= 2:
        sca[sup - 2].wait()
      sca[sup - 1].wait()
      return carry

    lax.fori_loop(0, n_super, superchunk, 0)
    plsc.subcore_barrier()
    pltpu.sync_copy(acc_sh.at[pl.ds(s * spr, spr)],
                    out_h.at[c, pl.ds(s * spr, spr)])

  args = [table, idx3, dst3]
  if scale:
    args.append(ew3)
  if with_ea:
    args.append(ea4)
  return k(*args)


def _tc_conv_block(p, a, hn, fp, fsp, Wm, Wa, Wn, Wo, bo, Ws, bs, *, head):
  n = p.shape[1]
  din = p.shape[2]
  bn = n // 8
  grid = (n // bn,)

  row = lambda shp: pl.BlockSpec(shp, lambda i: (i, 0))
  full = lambda shp: pl.BlockSpec(shp, lambda i: (0, 0))
  in_specs = [
      pl.BlockSpec((1, bn, din), lambda i: (0, i, 0)),
      pl.BlockSpec((1, bn, din), lambda i: (1, i, 0)),
      row((bn, hn.shape[1])),
  ]
  args = [p, p, hn]
  if not head:
    in_specs += [
        pl.BlockSpec((1, bn, _C), lambda i: (0, i, 0)),
        pl.BlockSpec((1, bn, _C), lambda i: (1, i, 0)),
        row((bn, _C)), row((bn, _C)),
    ]
    args += [a, a, fp, fsp]
  if not head:
    in_specs += [full(Wm.shape), full(Wa.shape)]
    args += [Wm, Wa]
  else:
    in_specs += [full(Wm.shape)]
    args += [Wm]
  in_specs += [full(Wn.shape), full(Wo.shape), full((1, _C)),
               full(Ws.shape), full((1, _C))]
  args += [Wn, Wo, bo.reshape(1, _C), Ws, bs.reshape(1, _C)]

  def body(*refs):
    if head:
      (p0, p1, hn_r, W0_r, Wn_r, Wo_r, bo_r, Ws_r, bs_r, f_o, fs_o) = refs
      g = p0[0] + p1[0]
      agg = jnp.dot(g, W0_r[...], preferred_element_type=jnp.float32)
    else:
      (p0, p1, hn_r, a0, a1, fp_r, fsp_r, Wm_r, Wa_r, Wn_r, Wo_r, bo_r,
       Ws_r, bs_r, f_o, fs_o) = refs
      g = p0[0] + p1[0]
      av = a0[0] + a1[0]
      agg = (jnp.dot(g, Wm_r[...], preferred_element_type=jnp.float32)
             + jnp.dot(av, Wa_r[...], preferred_element_type=jnp.float32))
    node = jnp.dot(hn_r[...], Wn_r[...], preferred_element_type=jnp.float32)
    f = jnp.maximum(
        jnp.dot(agg, Wo_r[...], preferred_element_type=jnp.float32)
        + node + bo_r[...], 0.0)
    fs = jnp.maximum(
        jnp.dot(agg + node, Ws_r[...], preferred_element_type=jnp.float32)
        + bs_r[...], 0.0)
    if not head:
      f = f + fp_r[...]
      fs = fs + fsp_r[...]
    f_o[...] = f
    fs_o[...] = fs

  return pl.pallas_call(
      body,
      grid=grid,
      in_specs=in_specs,
      out_specs=[row((bn, _C)), row((bn, _C))],
      out_shape=[jax.ShapeDtypeStruct((n, _C), jnp.float32)] * 2,
  )(*args)


def _tc_fusion(f5, f6, W, b):
  n = f5.shape[0]
  bn = n // 8
  row = lambda shp: pl.BlockSpec(shp, lambda i: (i, 0))
  full = lambda shp: pl.BlockSpec(shp, lambda i: (0, 0))

  def body(f5_r, f6_r, W_r, b_r, o_r):
    fcat = jnp.concatenate([f5_r[...], f6_r[...]], axis=1)
    o_r[...] = jnp.maximum(
        jnp.dot(fcat, W_r[...], preferred_element_type=jnp.float32)
        + b_r[...], 0.0)

  return pl.pallas_call(
      body,
      grid=(n // bn,),
      in_specs=[row((bn, _C)), row((bn, _C)), full(W.shape), full((1, 1024))],
      out_specs=row((bn, 1024)),
      out_shape=jax.ShapeDtypeStruct((n, 1024), jnp.float32),
  )(f5, f6, W, b.reshape(1, 1024))


def _tc_super(p5, p6, pc, W, b):
  def body(p50, p51, p60, p61, pc0, pc1, W_r, b_r, mean_o, fus_o):
    cnt = jnp.maximum(pc0[0][:, 0:1] + pc1[0][:, 0:1], 1.0)
    inv = 1.0 / cnt
    m5 = (p50[0] + p51[0]) * inv
    m6 = (p60[0] + p61[0]) * inv
    mean = jnp.concatenate([m5, m6], axis=1)
    mean_o[...] = mean
    fus_o[...] = jnp.maximum(
        jnp.dot(mean, W_r[...], preferred_element_type=jnp.float32)
        + b_r[...], 0.0)

  part = lambda cc: pl.BlockSpec((1, _NSUPER, _C), lambda i, cc=cc: (cc, 0, 0))
  full = lambda shp: pl.BlockSpec(shp, lambda i: (0,) * len(shp))
  return pl.pallas_call(
      body,
      grid=(1,),
      in_specs=[part(0), part(1), part(0), part(1), part(0), part(1),
                full(W.shape), full((1, 1024))],
      out_specs=[
          full((_NSUPER, 256)),
          full((_NSUPER, 1024)),
      ],
      out_shape=[
          jax.ShapeDtypeStruct((_NSUPER, 256), jnp.float32),
          jax.ShapeDtypeStruct((_NSUPER, 1024), jnp.float32),
      ],
  )(p5, p5, p6, p6, pc, pc, W, b.reshape(1, 1024))


def _pad_rows(v, r):
  return jnp.pad(v, ((0, r - v.shape[0]),) + ((0, 0),) * (v.ndim - 1))


def kernel(x, edges, edge_weights, edge_attrs, bbox_idx, head_Wm, head_Wa,
           head_Wn, head_Wo, head_bo, head_Ws, head_bs, blk_Wm, blk_Wa,
           blk_Wn, blk_Wo, blk_bo, blk_Ws, blk_bs, fus_W, fus_b, fuss_W,
           fuss_b):
  src = edges[0, 0]
  dst = edges[0, 1]
  ew = edge_weights[0]
  ea = edge_attrs[0]

  gran = _NW * _K * _SUP
  epad = ((_E + gran - 1) // gran) * gran
  src3 = _pad_rows(src, epad).reshape(_NW, -1, _K)
  dst3 = _pad_rows(dst, epad).reshape(_NW, -1, _K)
  ew3 = _pad_rows(ew, epad).reshape(_NW, -1, _K)

  x_pad128 = _pad_rows(jnp.pad(x, ((0, 0), (0, _C - x.shape[1]))), _NP)
  x_pad16 = x_pad128[:, :16]
  ea4 = _pad_rows(jnp.pad(ea, ((0, 0), (0, 16 - ea.shape[1]))),
                  epad).reshape(_NW, -1, 16, _C)

  ga_p = _sc_segsum(x_pad128, src3, dst3, ew3, ea4, n_out=_NP, d=_C,
                    scale=True, with_ea=True)

  w0 = (jnp.pad(head_Wm, ((0, _C - head_Wm.shape[0]), (0, 0)))
        + jnp.pad(head_Wa, ((16, _C - 16 - head_Wa.shape[0]), (0, 0))))
  wn0 = jnp.pad(head_Wn, ((0, 16 - head_Wn.shape[0]), (0, 0)))
  f, fs = _tc_conv_block(ga_p, None, x_pad16, None, None, w0, None, wn0,
                         head_Wo, head_bo, head_Ws, head_bs, head=True)

  wa_e = jnp.pad(blk_Wa, ((0, 0), (16, _C - 16 - blk_Wa.shape[1]), (0, 0)))
  feats = [f]
  feats_super = [fs]
  for i in range(6):
    g_p = _sc_segsum(feats[-1], src3, dst3, ew3, n_out=_NP, d=_C, scale=True)
    nf, nfs = _tc_conv_block(
        g_p, ga_p, feats_super[-1], feats[-1], feats_super[-1],
        blk_Wm[i], wa_e[i], blk_Wn[i], blk_Wo[i], blk_bo[i], blk_Ws[i],
        blk_bs[i], head=False)
    feats.append(nf)
    feats_super.append(nfs)

  f5, f6 = feats[5], feats[6]
  fs5, fs6 = feats_super[5], feats_super[6]

  fusion = _tc_fusion(f5, f6, fus_W, fus_b)
  out_feat = jnp.concatenate([fusion[:_N], f5[:_N], f6[:_N]], axis=1)

  npad = ((_N + _NW * _K - 1) // (_NW * _K)) * (_NW * _K)
  nsp = 1152
  bidx3 = jnp.pad(bbox_idx.astype(jnp.int32), (0, npad - _N),
                  constant_values=_NSUPER).reshape(_NW, -1, _K)
  niota3 = jnp.minimum(jnp.arange(npad, dtype=jnp.int32),
                       _N - 1).reshape(_NW, -1, _K)
  ones_tab = jnp.ones((_N, _C), jnp.float32)
  p5 = _sc_segsum(fs5, niota3, bidx3, n_out=nsp, d=_C, scale=False)
  p6 = _sc_segsum(fs6, niota3, bidx3, n_out=nsp, d=_C, scale=False)
  pc = _sc_segsum(ones_tab, niota3, bidx3, n_out=nsp, d=_C, scale=False)

  mean, fusion_s = _tc_super(p5, p6, pc, fuss_W, fuss_b)
  out_feat_super = jnp.concatenate([fusion_s, mean], axis=1)
  return (out_feat, out_feat_super)
